# baseline (speedup 1.0000x reference)
import functools

import jax
import jax.numpy as jnp
from jax import lax
from jax.experimental import pallas as pl
from jax.experimental.pallas import tpu as pltpu
from jax.experimental.pallas import tpu_sc as plsc

G = 8
BT = 256
BN = 1024

_NC = 2
_NS = 16
_NW = _NC * _NS


def _tile_group(cumt, t):
    g = 0
    for d in range(G):
        g = g + jnp.where(cumt[d] <= t, 1, 0)
    return jnp.minimum(g, G - 1)


def _grouped_body(cumt, x_ref, qw0, qw1, s0, s1, bw, bias_ref, out_ref,
                  xbf, wbf):
    c = pl.program_id(0)
    t = pl.program_id(1)
    nc_s = pl.num_programs(0) // 2
    g = _tile_group(cumt, t)
    prev_g = _tile_group(cumt, jnp.maximum(t - 1, 0))
    new_g = (t == 0) | (g != prev_g)

    @pl.when((c == 0) & (t == 0))
    def _():
        xbf[...] = x_ref[...].astype(jnp.bfloat16)

    @pl.when(new_g & (c < nc_s))
    def _():
        scale = s0[0, 0, 0, :]
        wbf[...] = (bw[...] + scale[:, None] *
                    (qw0[0] - 8).astype(jnp.float32)).astype(jnp.bfloat16)

    @pl.when(new_g & (c >= nc_s))
    def _():
        scale = s1[0, 0, 0, :]
        wbf[...] = (bw[...] + scale[:, None] *
                    (qw1[0] - 8).astype(jnp.float32)).astype(jnp.bfloat16)

    xb = xbf[pl.ds(t * BT, BT), :]
    out_ref[...] = lax.dot_general(xb, wbf[...], (((1,), (1,)), ((), ())),
                                   preferred_element_type=jnp.float32
                                   ) + bias_ref[0]


def _sc_row_gather(table, idx, n_chunks, nbuf):
    B = idx.shape[0]
    D = table.shape[1]
    b_per_w = B // _NW
    b_per_c = b_per_w // n_chunks
    mesh = plsc.VectorSubcoreMesh(core_axis_name="c", subcore_axis_name="s")


    @functools.partial(
        pl.kernel, mesh=mesh,
        out_type=jax.ShapeDtypeStruct((B, D), table.dtype),
        scratch_types=(
            [pltpu.VMEM((b_per_w,), jnp.int32)] +
            [pltpu.VMEM((b_per_c, D), table.dtype) for _ in range(nbuf)] +
            [pltpu.SemaphoreType.DMA for _ in range(2 * nbuf)]
        ),
    )
    def k(table_hbm, idx_hbm, out_hbm, idx_all, *bufs_sems):
        bufs = bufs_sems[:nbuf]
        gsem = bufs_sems[nbuf:2 * nbuf]
        ssem = bufs_sems[2 * nbuf:]
        wid = lax.axis_index("s") * _NC + lax.axis_index("c")
        base = wid * b_per_w
        pltpu.sync_copy(idx_hbm.at[pl.ds(base, b_per_w)], idx_all)
        gathers = [None] * n_chunks
        stores = [None] * n_chunks

        def _drain(d):
            gathers[d].wait()
            stores[d] = pltpu.async_copy(
                bufs[d % nbuf],
                out_hbm.at[pl.ds(base + d * b_per_c, b_per_c)],
                ssem[d % nbuf])

        for ch in range(n_chunks):
            b = ch % nbuf
            if ch >= nbuf:
                stores[ch - nbuf].wait()
            gathers[ch] = pltpu.async_copy(
                table_hbm.at[idx_all.at[pl.ds(ch * b_per_c, b_per_c)]],
                bufs[b], gsem[b])
            if ch >= nbuf - 1:
                _drain(ch - (nbuf - 1))
        for d in range(max(0, n_chunks - (nbuf - 1)), n_chunks):
            _drain(d)
        for d in range(max(0, n_chunks - nbuf), n_chunks):
            stores[d].wait()

    return k(table, idx)


@jax.jit
def kernel(x, base_W, bias, qweight0, qweight1, scales0, scales1, indices):
    T, D = x.shape
    NOUT = base_W.shape[0]
    SL = NOUT // 2
    nc = NOUT // BN
    nc_s = SL // BN
    TP = T + G * BT
    ntp = TP // BT

    onehot = (indices[:, None] == jnp.arange(G)[None, :]).astype(jnp.int32)
    sizes = jnp.sum(onehot, axis=0)
    rank = (jnp.cumsum(onehot, axis=0) - onehot)[jnp.arange(T), indices]
    tiles_g = jnp.maximum((sizes + BT - 1) // BT, 1)
    cumt = jnp.cumsum(tiles_g).astype(jnp.int32)
    pad_off = (jnp.concatenate([jnp.zeros(1, jnp.int32), cumt[:-1]]) * BT)
    pos = pad_off[indices] + rank
    src = jnp.zeros((TP,), jnp.int32).at[pos].set(
        jnp.arange(T, dtype=jnp.int32), unique_indices=True)

    x_p = _sc_row_gather(x, src, n_chunks=8, nbuf=4)

    s0r = scales0.reshape(G, nc_s, 1, BN)
    s1r = scales1.reshape(G, nc_s, 1, BN)
    bias_r = bias.reshape(nc, 1, BN)

    grid_spec = pltpu.PrefetchScalarGridSpec(
        num_scalar_prefetch=1,
        grid=(nc, ntp),
        in_specs=[
            pl.BlockSpec((TP, D), lambda c, t, cumt: (0, 0)),
            pl.BlockSpec((1, BN, D),
                         lambda c, t, cumt: (
                             jnp.where(c < nc_s, _tile_group(cumt, t), 0),
                             jnp.where(c < nc_s, c, 0), 0)),
            pl.BlockSpec((1, BN, D),
                         lambda c, t, cumt: (
                             jnp.where(c >= nc_s, _tile_group(cumt, t), 0),
                             jnp.where(c >= nc_s, c - nc_s, 0), 0)),
            pl.BlockSpec((1, 1, 1, BN),
                         lambda c, t, cumt: (
                             jnp.where(c < nc_s, _tile_group(cumt, t), 0),
                             jnp.where(c < nc_s, c, 0), 0, 0)),
            pl.BlockSpec((1, 1, 1, BN),
                         lambda c, t, cumt: (
                             jnp.where(c >= nc_s, _tile_group(cumt, t), 0),
                             jnp.where(c >= nc_s, c - nc_s, 0), 0, 0)),
            pl.BlockSpec((BN, D), lambda c, t, cumt: (c, 0)),
            pl.BlockSpec((1, 1, BN), lambda c, t, cumt: (c, 0, 0)),
        ],
        out_specs=pl.BlockSpec((BT, BN), lambda c, t, cumt: (t, c)),
        scratch_shapes=[
            pltpu.VMEM((TP, D), jnp.bfloat16),
            pltpu.VMEM((BN, D), jnp.bfloat16),
        ],
    )

    out_p = pl.pallas_call(
        _grouped_body,
        grid_spec=grid_spec,
        out_shape=jax.ShapeDtypeStruct((TP, NOUT), jnp.float32),
        compiler_params=pltpu.CompilerParams(
            dimension_semantics=("arbitrary", "arbitrary")),
    )(cumt, x_p, qweight0, qweight1, s0r, s1r, base_W, bias_r)

    return _sc_row_gather(out_p, pos, n_chunks=8, nbuf=3)


# speedup vs baseline: 1.3407x; 1.3407x over previous
import functools

import jax
import jax.numpy as jnp
from jax import lax
from jax.experimental import pallas as pl
from jax.experimental.pallas import tpu as pltpu
from jax.experimental.pallas import tpu_sc as plsc

G = 8
_NC = 2
_NS = 16
_NW = _NC * _NS
BT = 256
BN = 1024


def _grouped_body(rows, grps, firsts, starts, ends,
                  x_ref, qw0, qw1, s0, s1, bw, bias_ref, out_ref,
                  xbf, wbf):
    c = pl.program_id(0)
    w = pl.program_id(1)
    g = grps[w]
    r = rows[w]
    first = firsts[w]
    start = starts[w]
    end = ends[w]
    nc_s = pl.num_programs(0) // 2

    @pl.when((c == 0) & (w == 0))
    def _():
        xbf[...] = x_ref[...].astype(jnp.bfloat16)

    prev_g = grps[jnp.maximum(w - 1, 0)]
    new_w = (w == 0) | (g != prev_g)

    @pl.when(new_w & (c < nc_s))
    def _():
        scale = s0[0, 0, 0, :]
        wbf[...] = (bw[...] + scale[:, None] *
                    (qw0[0] - 8).astype(jnp.float32)).astype(jnp.bfloat16)

    @pl.when(new_w & (c >= nc_s))
    def _():
        scale = s1[0, 0, 0, :]
        wbf[...] = (bw[...] + scale[:, None] *
                    (qw1[0] - 8).astype(jnp.float32)).astype(jnp.bfloat16)

    row_ids = r * BT + lax.broadcasted_iota(jnp.int32, (BT, 1), 0)
    mask = (row_ids >= start) & (row_ids < end)
    xb = jnp.where(mask, xbf[pl.ds(r * BT, BT), :], jnp.bfloat16(0))
    contrib = lax.dot_general(xb, wbf[...], (((1,), (1,)), ((), ())),
                              preferred_element_type=jnp.float32)

    @pl.when(first == 1)
    def _():
        out_ref[...] = contrib + bias_ref[0]

    @pl.when(first == 0)
    def _():
        out_ref[...] += contrib


def _routing_metadata(sizes, T, nt):
    W = nt + G - 1
    off = jnp.concatenate([jnp.zeros(1, jnp.int32), jnp.cumsum(sizes)])
    start_t = off[:-1] // BT
    end_t = jnp.where(sizes > 0, (off[1:] - 1) // BT, start_t - 1)
    tiles = jnp.maximum(end_t - start_t + 1, 0)
    cum = jnp.cumsum(tiles)
    wids = jnp.arange(W, dtype=jnp.int32)
    gid = jnp.searchsorted(cum, wids, side='right').astype(jnp.int32)
    gid_c = jnp.minimum(gid, G - 1)
    prev_cum = jnp.where(gid_c > 0, cum[gid_c - 1], 0)
    rid = start_t[gid_c] + (wids - prev_cum)
    valid = wids < cum[-1]
    rid = jnp.where(valid, rid, nt - 1).astype(jnp.int32)
    gcl = jnp.where(valid, gid_c, G - 1).astype(jnp.int32)
    st = jnp.where(valid, off[gcl], 0).astype(jnp.int32)
    en = jnp.where(valid, off[gcl + 1], 0).astype(jnp.int32)
    first = jnp.concatenate([jnp.ones(1, jnp.int32),
                             (rid[1:] != rid[:-1]).astype(jnp.int32)])
    return rid, gcl, first, st, en, off


def _sc_row_gather(table, idx, n_chunks, nbuf):
    B = idx.shape[0]
    D = table.shape[1]
    b_per_w = B // _NW
    b_per_c = b_per_w // n_chunks
    mesh = plsc.VectorSubcoreMesh(core_axis_name="c", subcore_axis_name="s")


    @functools.partial(
        pl.kernel, mesh=mesh,
        out_type=jax.ShapeDtypeStruct((B, D), table.dtype),
        scratch_types=(
            [pltpu.VMEM((b_per_w,), jnp.int32)] +
            [pltpu.VMEM((b_per_c, D), table.dtype) for _ in range(nbuf)] +
            [pltpu.SemaphoreType.DMA for _ in range(2 * nbuf)]
        ),
    )
    def k(table_hbm, idx_hbm, out_hbm, idx_all, *bufs_sems):
        bufs = bufs_sems[:nbuf]
        gsem = bufs_sems[nbuf:2 * nbuf]
        ssem = bufs_sems[2 * nbuf:]
        wid = lax.axis_index("s") * _NC + lax.axis_index("c")
        base = wid * b_per_w
        pltpu.sync_copy(idx_hbm.at[pl.ds(base, b_per_w)], idx_all)
        gathers = [None] * n_chunks
        stores = [None] * n_chunks

        def _drain(d):
            gathers[d].wait()
            stores[d] = pltpu.async_copy(
                bufs[d % nbuf],
                out_hbm.at[pl.ds(base + d * b_per_c, b_per_c)],
                ssem[d % nbuf])

        for ch in range(n_chunks):
            b = ch % nbuf
            if ch >= nbuf:
                stores[ch - nbuf].wait()
            gathers[ch] = pltpu.async_copy(
                table_hbm.at[idx_all.at[pl.ds(ch * b_per_c, b_per_c)]],
                bufs[b], gsem[b])
            if ch >= nbuf - 1:
                _drain(ch - (nbuf - 1))
        for d in range(max(0, n_chunks - (nbuf - 1)), n_chunks):
            _drain(d)
        for d in range(max(0, n_chunks - nbuf), n_chunks):
            stores[d].wait()

    return k(table, idx)


@jax.jit
def kernel(x, base_W, bias, qweight0, qweight1, scales0, scales1, indices):
    T, D = x.shape
    NOUT = base_W.shape[0]
    SL = NOUT // 2
    nt = T // BT
    W = nt + G - 1
    nc = NOUT // BN
    nc_s = SL // BN

    onehot = (indices[:, None] == jnp.arange(G)[None, :]).astype(jnp.int32)
    sizes = jnp.sum(onehot, axis=0)
    rank = (jnp.cumsum(onehot, axis=0) - onehot)[jnp.arange(T), indices]
    rid, gcl, first, st, en, off = _routing_metadata(sizes, T, nt)
    pos = off[indices] + rank
    src = jnp.zeros((T,), jnp.int32).at[pos].set(
        jnp.arange(T, dtype=jnp.int32), unique_indices=True)
    x_s = _sc_row_gather(x, src, n_chunks=4, nbuf=4)

    s0r = scales0.reshape(G, nc_s, 1, BN)
    s1r = scales1.reshape(G, nc_s, 1, BN)
    bias_r = bias.reshape(nc, 1, BN)

    grid_spec = pltpu.PrefetchScalarGridSpec(
        num_scalar_prefetch=5,
        grid=(nc, W),
        in_specs=[
            pl.BlockSpec((T, D), lambda c, w, *s: (0, 0)),
            pl.BlockSpec((1, BN, D),
                         lambda c, w, rows, grps, *s: (
                             jnp.where(c < nc_s, grps[w], 0),
                             jnp.where(c < nc_s, c, 0), 0)),
            pl.BlockSpec((1, BN, D),
                         lambda c, w, rows, grps, *s: (
                             jnp.where(c >= nc_s, grps[w], 0),
                             jnp.where(c >= nc_s, c - nc_s, 0), 0)),
            pl.BlockSpec((1, 1, 1, BN),
                         lambda c, w, rows, grps, *s: (
                             jnp.where(c < nc_s, grps[w], 0),
                             jnp.where(c < nc_s, c, 0), 0, 0)),
            pl.BlockSpec((1, 1, 1, BN),
                         lambda c, w, rows, grps, *s: (
                             jnp.where(c >= nc_s, grps[w], 0),
                             jnp.where(c >= nc_s, c - nc_s, 0), 0, 0)),
            pl.BlockSpec((BN, D), lambda c, w, *s: (c, 0)),
            pl.BlockSpec((1, 1, BN), lambda c, w, *s: (c, 0, 0)),
        ],
        out_specs=pl.BlockSpec((BT, BN), lambda c, w, rows, *s: (rows[w], c)),
        scratch_shapes=[
            pltpu.VMEM((T, D), jnp.bfloat16),
            pltpu.VMEM((BN, D), jnp.bfloat16),
        ],
    )

    out_s = pl.pallas_call(
        _grouped_body,
        grid_spec=grid_spec,
        out_shape=jax.ShapeDtypeStruct((T, NOUT), jnp.float32),
        compiler_params=pltpu.CompilerParams(
            dimension_semantics=("arbitrary", "arbitrary")),
    )(rid, gcl, first, st, en,
      x_s, qweight0, qweight1, s0r, s1r, base_W, bias_r)

    return jnp.take(out_s, pos, axis=0)
